# baseline (speedup 1.0000x reference)
import functools

import jax
import jax.numpy as jnp
from jax import lax
from jax.experimental import pallas as pl
from jax.experimental.pallas import tpu as pltpu
from jax.experimental.pallas import tpu_sc as plsc

B = 16384
D = 32
NC = 2
NS = 16
NW = NC * NS
B_PER_W = B // NW
CHUNK = 128
N_CHUNK = B_PER_W // CHUNK

_mesh = plsc.VectorSubcoreMesh(core_axis_name="c", subcore_axis_name="s")


@functools.partial(
    pl.kernel,
    out_type=jax.ShapeDtypeStruct((B, D), jnp.float32),
    mesh=_mesh,
    compiler_params=pltpu.CompilerParams(use_tc_tiling_on_sc=False),
    scratch_types=[
        pltpu.VMEM((N_CHUNK, CHUNK), jnp.int32),
        pltpu.VMEM((N_CHUNK, CHUNK), jnp.int32),
        pltpu.VMEM((B_PER_W, D), jnp.float32),
        pltpu.VMEM((B_PER_W, D), jnp.float32),
        pltpu.SemaphoreType.DMA,
    ],
)
def _gather_mul(uidx_hbm, iidx_hbm, utab_hbm, itab_hbm, out_hbm,
                uidx_v, iidx_v, urows_v, irows_v, sem):
    wid = lax.axis_index("s") * NC + lax.axis_index("c")
    base = wid * B_PER_W
    pltpu.sync_copy(uidx_hbm.at[wid], uidx_v)
    pltpu.sync_copy(iidx_hbm.at[wid], iidx_v)
    copies = []
    for j in range(N_CHUNK):
        sl = pl.ds(j * CHUNK, CHUNK)
        copies.append(pltpu.async_copy(utab_hbm.at[uidx_v.at[j]], urows_v.at[sl], sem))
        copies.append(pltpu.async_copy(itab_hbm.at[iidx_v.at[j]], irows_v.at[sl], sem))
    for c in copies:
        c.wait()

    def body(r, carry):
        for h in range(D // 16):
            sl = pl.ds(h * 16, 16)
            urows_v[r, sl] = urows_v[r, sl] * irows_v[r, sl]
        return carry

    lax.fori_loop(0, B_PER_W, body, 0)
    pltpu.sync_copy(urows_v, out_hbm.at[pl.ds(base, B_PER_W)])


def _mlp_body(x_ref, wa_ref, ba_ref, wo_ref, bo_ref, o_ref):
    x = x_ref[...]
    h = jnp.maximum(
        jnp.dot(x, wa_ref[...], preferred_element_type=jnp.float32) + ba_ref[...],
        0.0)
    o_ref[...] = jax.nn.sigmoid(
        jnp.dot(h, wo_ref[...], preferred_element_type=jnp.float32) + bo_ref[...])


_GRID = 8
_BLK = B // _GRID


def _mlp(x, W_att, b_att2, W_out, b_out2):
    return pl.pallas_call(
        _mlp_body,
        grid=(_GRID,),
        in_specs=[
            pl.BlockSpec((_BLK, D), lambda i: (i, 0)),
            pl.BlockSpec((D, D), lambda i: (0, 0)),
            pl.BlockSpec((1, D), lambda i: (0, 0)),
            pl.BlockSpec((D, 1), lambda i: (0, 0)),
            pl.BlockSpec((1, 1), lambda i: (0, 0)),
        ],
        out_specs=pl.BlockSpec((_BLK, 1), lambda i: (i, 0)),
        out_shape=jax.ShapeDtypeStruct((B, 1), jnp.float32),
    )(x, W_att, b_att2, W_out, b_out2)


def kernel(user_input, item_input, user_table, item_table, W_att, b_att, W_out, b_out):
    uidx = user_input.astype(jnp.int32).reshape(NW, N_CHUNK, CHUNK)
    iidx = item_input.astype(jnp.int32).reshape(NW, N_CHUNK, CHUNK)
    x = _gather_mul(uidx, iidx, user_table, item_table)
    return _mlp(x, W_att, b_att.reshape(1, D), W_out, b_out.reshape(1, 1))


# speedup vs baseline: 3.3633x; 3.3633x over previous
import functools

import jax
import jax.numpy as jnp
from jax import lax
from jax.experimental import pallas as pl
from jax.experimental.pallas import tpu as pltpu
from jax.experimental.pallas import tpu_sc as plsc

B = 16384
D = 32
V = 1_000_000
NC = 2
NS = 16
NW = NC * NS
B_PER_W = B // NW
N_CHUNK = B_PER_W // 16

RING = 8

_mesh = plsc.VectorSubcoreMesh(core_axis_name="c", subcore_axis_name="s")


@functools.partial(
    pl.kernel,
    out_type=jax.ShapeDtypeStruct((B * D,), jnp.float32),
    mesh=_mesh,
    compiler_params=pltpu.CompilerParams(
        use_tc_tiling_on_sc=True, needs_layout_passes=False),
    scratch_types=[
        pltpu.VMEM((B_PER_W,), jnp.int32),
        pltpu.VMEM((B_PER_W,), jnp.int32),
        pltpu.VMEM((D, RING * 128), jnp.float32),
        pltpu.VMEM((D, RING * 128), jnp.float32),
        pltpu.VMEM((B_PER_W * D,), jnp.float32),
        pltpu.SemaphoreType.DMA,
        pltpu.SemaphoreType.DMA,
    ],
)
def _gather_mul(uidx_hbm, iidx_hbm, utab_hbm, itab_hbm, out_hbm,
                uidx_v, iidx_v, uring_v, iring_v, prod_v, sem_a, sem_b):
    wid = lax.axis_index("s") * NC + lax.axis_index("c")
    base = wid * B_PER_W
    pltpu.sync_copy(uidx_hbm.at[pl.ds(base, B_PER_W)], uidx_v)
    pltpu.sync_copy(iidx_hbm.at[pl.ds(base, B_PER_W)], iidx_v)

    iota16 = lax.iota(jnp.int32, 16)
    half = RING // 2

    def issue(chunk16, lo, slot0, sem):
        evu = uidx_v[pl.ds(chunk16, 16)]
        evi = iidx_v[pl.ds(chunk16, 16)]
        for j in range(half):
            bu = pl.multiple_of((evu[lo + j] >> 7) * 128, 128)
            bi = pl.multiple_of((evi[lo + j] >> 7) * 128, 128)
            sl = pl.ds((slot0 + j) * 128, 128)
            pltpu.async_copy(utab_hbm.at[:, pl.ds(bu, 128)], uring_v.at[:, sl], sem)
            pltpu.async_copy(itab_hbm.at[:, pl.ds(bi, 128)], iring_v.at[:, sl], sem)

    def drain(sem):
        for _ in range(2):
            pltpu.make_async_copy(
                utab_hbm.at[:, pl.ds(0, half * 128)],
                uring_v.at[:, pl.ds(0, half * 128)], sem).wait()

    def extract(chunk16, lo, slot0):
        evu = uidx_v[pl.ds(chunk16, 16)]
        evi = iidx_v[pl.ds(chunk16, 16)]
        for j in range(half):
            cu = (evu[lo + j] & 127) + (slot0 + j) * 128
            ci = (evi[lo + j] & 127) + (slot0 + j) * 128
            cuv = jnp.full((16,), cu, jnp.int32)
            civ = jnp.full((16,), ci, jnp.int32)
            for fh in range(2):
                rows = iota16 + fh * 16
                uvals = plsc.load_gather(uring_v, [rows, cuv])
                ivals = plsc.load_gather(iring_v, [rows, civ])
                prod_v[pl.ds((chunk16 + lo + j) * D + fh * 16, 16)] = uvals * ivals

    sems = (sem_a, sem_b)
    issue(0, 0, 0, sem_a)
    issue(0, half, half, sem_b)

    def pipe_body(p, carry):
        c16 = p * 16
        for q in range(4):
            sem = sems[q % 2]
            slot0 = (q % 2) * half
            drain(sem)
            extract(c16, q * half, slot0)
            nxt_chunk = c16 + 16 * ((q + 2) // 4)
            nxt_lo = half * ((q + 2) % 4)
            if q < 2:
                issue(nxt_chunk, nxt_lo, slot0, sem)
            else:
                @pl.when(p < (B_PER_W // 16) - 1)
                def _(nc=nxt_chunk, nl=nxt_lo, s0=slot0, sm=sem):
                    issue(nc, nl, s0, sm)
        return carry

    lax.fori_loop(0, B_PER_W // 16, pipe_body, 0)
    pltpu.sync_copy(prod_v, out_hbm.at[pl.ds(wid * (B_PER_W * D), B_PER_W * D)])


def _mlp_body(x_ref, wa_ref, ba_ref, wo_ref, bo_ref, o_ref):
    x = x_ref[...]
    h = jnp.maximum(
        jnp.dot(x, wa_ref[...], preferred_element_type=jnp.float32) + ba_ref[...],
        0.0)
    o_ref[...] = jax.nn.sigmoid(
        jnp.dot(h, wo_ref[...], preferred_element_type=jnp.float32) + bo_ref[...])


_GRID = 8
_BLK = B // _GRID


def _mlp(x, W_att, b_att2, W_out, b_out2):
    return pl.pallas_call(
        _mlp_body,
        grid=(_GRID,),
        in_specs=[
            pl.BlockSpec((_BLK, D), lambda i: (i, 0)),
            pl.BlockSpec((D, D), lambda i: (0, 0)),
            pl.BlockSpec((1, D), lambda i: (0, 0)),
            pl.BlockSpec((D, 1), lambda i: (0, 0)),
            pl.BlockSpec((1, 1), lambda i: (0, 0)),
        ],
        out_specs=pl.BlockSpec((_BLK, 1), lambda i: (i, 0)),
        out_shape=jax.ShapeDtypeStruct((B, 1), jnp.float32),
    )(x, W_att, b_att2, W_out, b_out2)


def kernel(user_input, item_input, user_table, item_table, W_att, b_att, W_out, b_out):
    uidx = user_input.astype(jnp.int32)
    iidx = item_input.astype(jnp.int32)
    flat = _gather_mul(uidx, iidx, user_table.T, item_table.T)
    x = flat.reshape(B, D)
    return _mlp(x, W_att, b_att.reshape(1, D), W_out, b_out.reshape(1, 1))


# speedup vs baseline: 3.5107x; 1.0438x over previous
import functools

import jax
import jax.numpy as jnp
from jax import lax
from jax.experimental import pallas as pl
from jax.experimental.pallas import tpu as pltpu
from jax.experimental.pallas import tpu_sc as plsc

B = 16384
D = 32
V = 1_000_000
NC = 2
NS = 16
NW = NC * NS
B_PER_W = B // NW
N_CHUNK = B_PER_W // 16

RING = 8

_mesh = plsc.VectorSubcoreMesh(core_axis_name="c", subcore_axis_name="s")


@functools.partial(
    pl.kernel,
    out_type=jax.ShapeDtypeStruct((B * D,), jnp.float32),
    mesh=_mesh,
    compiler_params=pltpu.CompilerParams(
        use_tc_tiling_on_sc=True, needs_layout_passes=False),
    scratch_types=[
        pltpu.VMEM((B_PER_W,), jnp.int32),
        pltpu.VMEM((B_PER_W,), jnp.int32),
        pltpu.VMEM((D, RING * 128), jnp.float32),
        pltpu.VMEM((D, RING * 128), jnp.float32),
        pltpu.VMEM((B_PER_W * D,), jnp.float32),
        pltpu.SemaphoreType.DMA,
        pltpu.SemaphoreType.DMA,
    ],
)
def _gather_mul(uidx_hbm, iidx_hbm, utab_hbm, itab_hbm, out_hbm,
                uidx_v, iidx_v, uring_v, iring_v, prod_v, sem_a, sem_b):
    wid = lax.axis_index("s") * NC + lax.axis_index("c")
    base = wid * B_PER_W
    pltpu.sync_copy(uidx_hbm.at[pl.ds(base, B_PER_W)], uidx_v)
    pltpu.sync_copy(iidx_hbm.at[pl.ds(base, B_PER_W)], iidx_v)

    iota16 = lax.iota(jnp.int32, 16)
    half = RING // 2

    def issue(chunk16, lo, slot0, sem):
        evu = uidx_v[pl.ds(chunk16, 16)]
        evi = iidx_v[pl.ds(chunk16, 16)]
        for j in range(half):
            bu = pl.multiple_of((evu[lo + j] >> 7) * 128, 128)
            bi = pl.multiple_of((evi[lo + j] >> 7) * 128, 128)
            sl = pl.ds((slot0 + j) * 128, 128)
            pltpu.async_copy(utab_hbm.at[:, pl.ds(bu, 128)], uring_v.at[:, sl], sem)
            pltpu.async_copy(itab_hbm.at[:, pl.ds(bi, 128)], iring_v.at[:, sl], sem)

    def drain(sem):
        for _ in range(2):
            pltpu.make_async_copy(
                utab_hbm.at[:, pl.ds(0, half * 128)],
                uring_v.at[:, pl.ds(0, half * 128)], sem).wait()

    def extract(chunk16, lo, slot0):
        evu = uidx_v[pl.ds(chunk16, 16)]
        evi = iidx_v[pl.ds(chunk16, 16)]
        for j in range(half):
            cu = (evu[lo + j] & 127) + (slot0 + j) * 128
            ci = (evi[lo + j] & 127) + (slot0 + j) * 128
            cuv = jnp.full((16,), cu, jnp.int32)
            civ = jnp.full((16,), ci, jnp.int32)
            for fh in range(2):
                rows = iota16 + fh * 16
                uvals = plsc.load_gather(uring_v, [rows, cuv])
                ivals = plsc.load_gather(iring_v, [rows, civ])
                prod_v[pl.ds((chunk16 + lo + j) * D + fh * 16, 16)] = uvals * ivals

    sems = (sem_a, sem_b)
    issue(0, 0, 0, sem_a)
    issue(0, half, half, sem_b)

    def pipe_body(p, carry):
        c16 = p * 16
        for q in range(4):
            sem = sems[q % 2]
            slot0 = (q % 2) * half
            drain(sem)
            extract(c16, q * half, slot0)
            nxt_chunk = c16 + 16 * ((q + 2) // 4)
            nxt_lo = half * ((q + 2) % 4)
            if q < 2:
                issue(nxt_chunk, nxt_lo, slot0, sem)
            else:
                @pl.when(p < (B_PER_W // 16) - 1)
                def _(nc=nxt_chunk, nl=nxt_lo, s0=slot0, sm=sem):
                    issue(nc, nl, s0, sm)
        return carry

    lax.fori_loop(0, B_PER_W // 16, pipe_body, 0)
    pltpu.sync_copy(prod_v, out_hbm.at[pl.ds(wid * (B_PER_W * D), B_PER_W * D)])


_GRID = 8
_BLK = B // _GRID
_ROWS = _BLK * D // 128


def _mlp_body(x_ref, wa_ref, ba_ref, wo_ref, bo_ref, o_ref):
    x = x_ref[...].reshape(_ROWS, 128)
    h = jnp.maximum(
        jnp.dot(x, wa_ref[...], preferred_element_type=jnp.float32) + ba_ref[...],
        0.0)
    o_ref[...] = jax.nn.sigmoid(
        jnp.dot(h, wo_ref[...], preferred_element_type=jnp.float32) + bo_ref[...])


def _mlp(flat, W_blk, b_blk, Wo_blk, b_out2):
    return pl.pallas_call(
        _mlp_body,
        grid=(_GRID,),
        in_specs=[
            pl.BlockSpec((_BLK * D,), lambda i: (i,)),
            pl.BlockSpec((128, 128), lambda i: (0, 0)),
            pl.BlockSpec((1, 128), lambda i: (0, 0)),
            pl.BlockSpec((128, 4), lambda i: (0, 0)),
            pl.BlockSpec((1, 1), lambda i: (0, 0)),
        ],
        out_specs=pl.BlockSpec((_ROWS, 4), lambda i: (i, 0)),
        out_shape=jax.ShapeDtypeStruct((B // 4, 4), jnp.float32),
    )(flat, W_blk, b_blk, Wo_blk, b_out2)


def kernel(user_input, item_input, user_table, item_table, W_att, b_att, W_out, b_out):
    uidx = user_input.astype(jnp.int32)
    iidx = item_input.astype(jnp.int32)
    flat = _gather_mul(uidx, iidx, user_table.T, item_table.T)
    eye4 = jnp.eye(4, dtype=jnp.float32)
    W_blk = jnp.kron(eye4, W_att)
    Wo_blk = jnp.kron(eye4, W_out)
    b_blk = jnp.tile(b_att, 4).reshape(1, 128)
    out = _mlp(flat, W_blk, b_blk, Wo_blk, b_out.reshape(1, 1))
    return out.reshape(B, 1)
